# Initial kernel scaffold; baseline (speedup 1.0000x reference)
import jax
import jax.numpy as jnp
from jax.experimental import pallas as pl


def kernel(x, edge_index, W0, b0, W1, b1, Wl1, bl1, Wl2, bl2, Wl3, bl3):
    raise NotImplementedError("write your pallas kernel here")


# speedup vs baseline: 12.3414x; 12.3414x over previous
import functools

import jax
import jax.numpy as jnp
from jax import lax
from jax.experimental import pallas as pl
from jax.experimental.pallas import tpu as pltpu
from jax.experimental.pallas import tpu_sc as plsc

_N = 10000
_E = 320000
_NP = 10240
_NC = 2
_NS = 16
_CHUNK = 128
_EP = ((_E + _NC * _NS * _CHUNK - 1) // (_NC * _NS * _CHUNK)) * (_NC * _NS * _CHUNK)
_BR = 1024


def _mesh():
    return plsc.VectorSubcoreMesh(
        core_axis_name="c", subcore_axis_name="s", num_cores=_NC, num_subcores=_NS
    )


def _sc_deg(dst_p):
    ch = _EP // (_NC * _NS * _CHUNK)

    @functools.partial(
        pl.kernel,
        out_type=jax.ShapeDtypeStruct((_NC, _NP, 16), jnp.float32),
        mesh=_mesh(),
        scratch_types=[
            pltpu.VMEM((_CHUNK,), jnp.int32),
            pltpu.VMEM((_CHUNK, 16), jnp.float32),
            pltpu.VMEM((_CHUNK, 16), jnp.float32),
            pltpu.VMEM_SHARED((_NP, 16), jnp.float32),
        ],
    )
    def k(dst_hbm, deg_hbm, idx_v, ones_v, stage_v, acc_sh):
        c = lax.axis_index("c")
        s = lax.axis_index("s")
        tid = c * _NS + s
        zero16 = jnp.zeros((16,), jnp.float32)
        one16 = jnp.ones((16,), jnp.float32)

        @pl.loop(0, _CHUNK)
        def _(i):
            ones_v[i, :] = one16
            stage_v[i, :] = zero16

        for j in range(5):
            pltpu.sync_copy(stage_v, acc_sh.at[pl.ds(s * 640 + j * 128, 128)])
        plsc.subcore_barrier()

        base = tid * (ch * _CHUNK)

        @pl.loop(0, ch)
        def _(ci):
            pltpu.sync_copy(dst_hbm.at[pl.ds(base + ci * _CHUNK, _CHUNK)], idx_v)
            pltpu.sync_copy(ones_v, acc_sh.at[idx_v], add=True)

        plsc.subcore_barrier()
        for j in range(5):
            pltpu.sync_copy(acc_sh.at[pl.ds(s * 640 + j * 128, 128)], stage_v)
            pltpu.sync_copy(stage_v, deg_hbm.at[c, pl.ds(s * 640 + j * 128, 128)])

    return k(dst_p)


def _sc_agg(table, src_idx, dst_p, edge_split):
    ntile = _NC * _NS if edge_split else _NS
    ch = _EP // (ntile * _CHUNK)

    @functools.partial(
        pl.kernel,
        out_type=jax.ShapeDtypeStruct((_NC, _NP, 128), jnp.float32),
        mesh=_mesh(),
        scratch_types=[
            pltpu.VMEM((_CHUNK,), jnp.int32),
            pltpu.VMEM((_CHUNK,), jnp.int32),
            pltpu.VMEM((_CHUNK, 128), jnp.float32),
            pltpu.VMEM_SHARED((_NP, 128), jnp.float32),
            pltpu.SemaphoreType.DMA,
        ],
    )
    def k(table_hbm, src_hbm, dst_hbm, out_hbm, sidx, didx, rows, acc_sh, sem):
        c = lax.axis_index("c")
        s = lax.axis_index("s")
        zero16 = jnp.zeros((16,), jnp.float32)

        @pl.loop(0, _CHUNK)
        def _(i):
            for g in range(8):
                rows[i, pl.ds(g * 16, 16)] = zero16

        for j in range(5):
            pltpu.sync_copy(rows, acc_sh.at[pl.ds(s * 640 + j * 128, 128)])
        plsc.subcore_barrier()

        if edge_split:
            base = (c * _NS + s) * (ch * _CHUNK)
        else:
            base = s * (ch * _CHUNK)

        @pl.loop(0, ch)
        def _(ci):
            eb = base + ci * _CHUNK
            if edge_split:
                pltpu.sync_copy(src_hbm.at[pl.ds(eb, _CHUNK)], sidx)
            else:
                pltpu.sync_copy(src_hbm.at[pl.ds(c * _EP + eb, _CHUNK)], sidx)
            pltpu.sync_copy(dst_hbm.at[pl.ds(eb, _CHUNK)], didx)
            pltpu.async_copy(table_hbm.at[sidx], rows, sem).wait()
            pltpu.sync_copy(rows, acc_sh.at[didx], add=True)

        plsc.subcore_barrier()
        for j in range(5):
            pltpu.sync_copy(acc_sh.at[pl.ds(s * 640 + j * 128, 128)], rows)
            pltpu.sync_copy(rows, out_hbm.at[c, pl.ds(s * 640 + j * 128, 128)])

    return k(table, src_idx, dst_p)


def _tc_a(x_pad, w0, deg_a, deg_b):
    def body(x_ref, w_ref, da_ref, db_ref, t_ref, dinv_ref):
        deg = da_ref[:, :1] + db_ref[:, :1] + 1.0
        dinv = lax.rsqrt(deg)
        h = jnp.dot(x_ref[...], w_ref[...], preferred_element_type=jnp.float32)
        t_ref[...] = h * dinv
        dinv_ref[...] = dinv

    g = _NP // _BR
    return pl.pallas_call(
        body,
        grid=(g,),
        in_specs=[
            pl.BlockSpec((_BR, 128), lambda i: (i, 0)),
            pl.BlockSpec((128, 128), lambda i: (0, 0)),
            pl.BlockSpec((_BR, 16), lambda i: (i, 0)),
            pl.BlockSpec((_BR, 16), lambda i: (i, 0)),
        ],
        out_specs=[
            pl.BlockSpec((_BR, 128), lambda i: (i, 0)),
            pl.BlockSpec((_BR, 1), lambda i: (i, 0)),
        ],
        out_shape=[
            jax.ShapeDtypeStruct((_NP, 128), jnp.float32),
            jax.ShapeDtypeStruct((_NP, 1), jnp.float32),
        ],
    )(x_pad, w0, deg_a, deg_b)


def _tc_b(acc_a, acc_b, hs0, dinv, b0, w1):
    def body(aa_ref, ab_ref, h_ref, d_ref, b_ref, w_ref, o_ref):
        i = pl.program_id(0)
        dinv = d_ref[...]
        agg = aa_ref[...] + ab_ref[...] + h_ref[...]
        out0 = jnp.maximum(agg * dinv + b_ref[...], 0.0)
        h1 = jnp.dot(out0, w_ref[...], preferred_element_type=jnp.float32)
        hs1 = h1 * dinv
        row = lax.broadcasted_iota(jnp.int32, (_BR, 1), 0) + i * _BR
        o_ref[...] = jnp.where(row < _N, hs1, 0.0)

    g = _NP // _BR
    return pl.pallas_call(
        body,
        grid=(g,),
        in_specs=[
            pl.BlockSpec((_BR, 128), lambda i: (i, 0)),
            pl.BlockSpec((_BR, 128), lambda i: (i, 0)),
            pl.BlockSpec((_BR, 128), lambda i: (i, 0)),
            pl.BlockSpec((_BR, 1), lambda i: (i, 0)),
            pl.BlockSpec((1, 128), lambda i: (0, 0)),
            pl.BlockSpec((128, 256), lambda i: (0, 0)),
        ],
        out_specs=pl.BlockSpec((_BR, 256), lambda i: (i, 0)),
        out_shape=jax.ShapeDtypeStruct((_NP, 256), jnp.float32),
    )(acc_a, acc_b, hs0, dinv, b0, w1)


def _tc_c(acc_a, acc_b, hs1, dinv, b1, wl1, bl1, wl2, bl2, wl3, bl3):
    def body(aa_ref, ab_ref, h_ref, d_ref, b_ref, w1_ref, c1_ref, w2_ref,
             c2_ref, w3_ref, c3_ref, o_ref):
        dinv = d_ref[...]
        agg = jnp.concatenate([aa_ref[...], ab_ref[...]], axis=1) + h_ref[...]
        h = jnp.maximum(agg * dinv + b_ref[...], 0.0)
        h = jnp.maximum(
            jnp.dot(h, w1_ref[...], preferred_element_type=jnp.float32)
            + c1_ref[...], 0.0)
        h = jnp.maximum(
            jnp.dot(h, w2_ref[...], preferred_element_type=jnp.float32)
            + c2_ref[...], 0.0)
        h = jnp.maximum(
            jnp.dot(h, w3_ref[...], preferred_element_type=jnp.float32)
            + c3_ref[...], 0.0)
        m = jnp.max(h, axis=1, keepdims=True)
        e = jnp.exp(h - m)
        o_ref[...] = e / jnp.sum(e, axis=1, keepdims=True)

    g = _NP // _BR
    return pl.pallas_call(
        body,
        grid=(g,),
        in_specs=[
            pl.BlockSpec((_BR, 128), lambda i: (i, 0)),
            pl.BlockSpec((_BR, 128), lambda i: (i, 0)),
            pl.BlockSpec((_BR, 256), lambda i: (i, 0)),
            pl.BlockSpec((_BR, 1), lambda i: (i, 0)),
            pl.BlockSpec((1, 256), lambda i: (0, 0)),
            pl.BlockSpec((256, 512), lambda i: (0, 0)),
            pl.BlockSpec((1, 512), lambda i: (0, 0)),
            pl.BlockSpec((512, 1024), lambda i: (0, 0)),
            pl.BlockSpec((1, 1024), lambda i: (0, 0)),
            pl.BlockSpec((1024, 64), lambda i: (0, 0)),
            pl.BlockSpec((1, 64), lambda i: (0, 0)),
        ],
        out_specs=pl.BlockSpec((_BR, 64), lambda i: (i, 0)),
        out_shape=jax.ShapeDtypeStruct((_NP, 64), jnp.float32),
    )(acc_a, acc_b, hs1, dinv, b1, wl1, bl1, wl2, bl2, wl3, bl3)


@jax.jit
def kernel(x, edge_index, W0, b0, W1, b1, Wl1, bl1, Wl2, bl2, Wl3, bl3):
    src = edge_index[0]
    dst = edge_index[1]
    npad = _EP - _E
    pad_idx = _N + (jnp.arange(npad, dtype=jnp.int32) % (_NP - _N))
    src_p = jnp.concatenate([src, pad_idx])
    dst_p = jnp.concatenate([dst, pad_idx])
    src2 = jnp.stack([src_p * 2, src_p * 2 + 1])

    x_pad = jnp.zeros((_NP, 128), jnp.float32).at[:_N].set(x)

    deg2 = _sc_deg(dst_p)
    deg_a = deg2[0]
    deg_b = deg2[1]

    hs0, dinv = _tc_a(x_pad, W0, deg_a, deg_b)
    acc0 = _sc_agg(hs0, src_p, dst_p, edge_split=True)
    hs1 = _tc_b(acc0[0], acc0[1], hs0, dinv, b0.reshape(1, 128), W1)
    acc1 = _sc_agg(hs1.reshape(2 * _NP, 128), src2.reshape(-1), dst_p,
                   edge_split=False)
    out = _tc_c(acc1[0], acc1[1], hs1, dinv, b1.reshape(1, 256),
                Wl1, bl1.reshape(1, 512), Wl2, bl2.reshape(1, 1024),
                Wl3, bl3.reshape(1, 64))
    return out[:_N]


# speedup vs baseline: 23.5147x; 1.9053x over previous
import functools

import jax
import jax.numpy as jnp
from jax import lax
from jax.experimental import pallas as pl
from jax.experimental.pallas import tpu as pltpu
from jax.experimental.pallas import tpu_sc as plsc

_N = 10000
_E = 320000
_NP = 10240
_NC = 2
_NS = 16
_CHUNK = 128
_IB = 8
_EP = _NC * _NS * _CHUNK * 80
_TAIL = _IB * _CHUNK
_BR = 1024


def _mesh():
    return plsc.VectorSubcoreMesh(
        core_axis_name="c", subcore_axis_name="s", num_cores=_NC, num_subcores=_NS
    )


def _sc_deg(dst_rows):
    ch = _EP // (_NC * _NS * _CHUNK)
    nsb = ch // _IB

    @functools.partial(
        pl.kernel,
        out_type=jax.ShapeDtypeStruct((_NC, _NP, 16), jnp.float32),
        mesh=_mesh(),
        scratch_types=[
            pltpu.VMEM((_IB, _CHUNK), jnp.int32),
            pltpu.VMEM((_CHUNK, 16), jnp.float32),
            pltpu.VMEM((_CHUNK, 16), jnp.float32),
            pltpu.VMEM_SHARED((_NP, 16), jnp.float32),
            pltpu.SemaphoreType.DMA,
        ],
    )
    def k(dst_hbm, deg_hbm, idx_v, ones_v, stage_v, acc_sh, ssem):
        c = lax.axis_index("c")
        s = lax.axis_index("s")
        tid = c * _NS + s
        zero16 = jnp.zeros((16,), jnp.float32)
        one16 = jnp.ones((16,), jnp.float32)

        @pl.loop(0, _CHUNK)
        def _(i):
            ones_v[i, :] = one16
            stage_v[i, :] = zero16

        for j in range(5):
            pltpu.sync_copy(stage_v, acc_sh.at[pl.ds(s * 640 + j * 128, 128)])
        plsc.subcore_barrier()

        tb = tid * ch
        pending = []
        for b in range(nsb):
            for d in pending:
                d.wait()
            pending = []
            pltpu.sync_copy(dst_hbm.at[pl.ds(tb + b * _IB, _IB)], idx_v)
            for j in range(_IB):
                pending.append(
                    pltpu.async_copy(ones_v, acc_sh.at[idx_v.at[j]], ssem,
                                     add=True))
        for d in pending:
            d.wait()

        plsc.subcore_barrier()
        for j in range(5):
            pltpu.sync_copy(acc_sh.at[pl.ds(s * 640 + j * 128, 128)], stage_v)
            pltpu.sync_copy(stage_v, deg_hbm.at[c, pl.ds(s * 640 + j * 128, 128)])

    return k(dst_rows)


def _sc_agg(table, src_idx, dst_p, edge_split):
    ntile = _NC * _NS if edge_split else _NS
    ch = _EP // (ntile * _CHUNK)
    nb = ch // (2 * _IB)

    @functools.partial(
        pl.kernel,
        out_type=jax.ShapeDtypeStruct((_NC, _NP, 128), jnp.float32),
        mesh=_mesh(),
        scratch_types=[
            pltpu.VMEM((2, _IB, _CHUNK), jnp.int32),
            pltpu.VMEM((2, _IB, _CHUNK), jnp.int32),
            pltpu.VMEM((2, _CHUNK, 128), jnp.float32),
            pltpu.VMEM_SHARED((_NP, 128), jnp.float32),
            pltpu.SemaphoreType.DMA,
            pltpu.SemaphoreType.DMA,
            pltpu.SemaphoreType.DMA,
            pltpu.SemaphoreType.DMA,
        ],
    )
    def k(table_hbm, src_hbm, dst_hbm, out_hbm, sidx, didx, rows, acc_sh,
          gsem0, gsem1, isem_s, isem_d):
        c = lax.axis_index("c")
        s = lax.axis_index("s")
        zero16 = jnp.zeros((16,), jnp.float32)
        gsems = (gsem0, gsem1)

        @pl.loop(0, _CHUNK)
        def _(i):
            for g in range(8):
                rows[0, i, pl.ds(g * 16, 16)] = zero16

        for j in range(5):
            pltpu.sync_copy(rows.at[0], acc_sh.at[pl.ds(s * 640 + j * 128, 128)])
        plsc.subcore_barrier()

        if edge_split:
            tb = (c * _NS + s) * ch
            soff = tb
        else:
            tb = s * ch
            soff = c * (_EP // _CHUNK) + tb

        def issue_load(t_rel, q):
            pltpu.async_copy(src_hbm.at[pl.ds(soff + t_rel, _IB)],
                             sidx.at[q], isem_s)
            pltpu.async_copy(dst_hbm.at[pl.ds(tb + t_rel, _IB)],
                             didx.at[q], isem_d)

        def drain_load(q):
            pltpu.make_async_copy(src_hbm.at[pl.ds(0, _IB)], sidx.at[q],
                                  isem_s).wait()
            pltpu.make_async_copy(dst_hbm.at[pl.ds(0, _IB)], didx.at[q],
                                  isem_d).wait()

        def gather(q, j, p):
            return pltpu.async_copy(table_hbm.at[sidx.at[q, j]], rows.at[p],
                                    gsems[p])

        issue_load(0, 0)

        @pl.loop(0, nb)
        def _(t):
            rel = t * (2 * _IB)
            drain_load(0)
            issue_load(rel + _IB, 1)
            gd = [None, None]
            gd[0] = gather(0, 0, 0)
            for jj in range(2 * _IB):
                q, j = divmod(jj, _IB)
                p = jj % 2
                nxt = jj + 1
                if nxt < 2 * _IB:
                    if nxt == _IB:
                        drain_load(1)
                    nq, nj = divmod(nxt, _IB)
                    gd[nxt % 2] = gather(nq, nj, nxt % 2)
                gd[p].wait()
                pltpu.sync_copy(rows.at[p], acc_sh.at[didx.at[q, j]], add=True)
                if nxt == _IB:
                    issue_load(rel + 2 * _IB, 0)

        drain_load(0)
        plsc.subcore_barrier()
        for j in range(5):
            pltpu.sync_copy(acc_sh.at[pl.ds(s * 640 + j * 128, 128)], rows.at[0])
            pltpu.sync_copy(rows.at[0], out_hbm.at[c, pl.ds(s * 640 + j * 128, 128)])

    return k(table, src_idx, dst_p)


def _tc_a1(x_pad, w0):
    def body(x_ref, w_ref, h_ref):
        h_ref[...] = jnp.dot(x_ref[...], w_ref[...],
                             preferred_element_type=jnp.float32)

    g = _NP // _BR
    return pl.pallas_call(
        body,
        grid=(g,),
        in_specs=[
            pl.BlockSpec((_BR, 128), lambda i: (i, 0)),
            pl.BlockSpec((128, 128), lambda i: (0, 0)),
        ],
        out_specs=pl.BlockSpec((_BR, 128), lambda i: (i, 0)),
        out_shape=jax.ShapeDtypeStruct((_NP, 128), jnp.float32),
    )(x_pad, w0)


def _tc_a2(h0, deg_a, deg_b):
    def body(h_ref, da_ref, db_ref, t_ref, dinv_ref):
        deg = da_ref[:, :1] + db_ref[:, :1] + 1.0
        dinv = lax.rsqrt(deg)
        t_ref[...] = h_ref[...] * dinv
        dinv_ref[...] = dinv

    g = _NP // _BR
    return pl.pallas_call(
        body,
        grid=(g,),
        in_specs=[
            pl.BlockSpec((_BR, 128), lambda i: (i, 0)),
            pl.BlockSpec((_BR, 16), lambda i: (i, 0)),
            pl.BlockSpec((_BR, 16), lambda i: (i, 0)),
        ],
        out_specs=[
            pl.BlockSpec((_BR, 128), lambda i: (i, 0)),
            pl.BlockSpec((_BR, 1), lambda i: (i, 0)),
        ],
        out_shape=[
            jax.ShapeDtypeStruct((_NP, 128), jnp.float32),
            jax.ShapeDtypeStruct((_NP, 1), jnp.float32),
        ],
    )(h0, deg_a, deg_b)


def _tc_b(acc_a, acc_b, hs0, dinv, b0, w1):
    def body(aa_ref, ab_ref, h_ref, d_ref, b_ref, w_ref, o_ref):
        i = pl.program_id(0)
        dinv = d_ref[...]
        agg = aa_ref[...] + ab_ref[...] + h_ref[...]
        out0 = jnp.maximum(agg * dinv + b_ref[...], 0.0)
        h1 = jnp.dot(out0, w_ref[...], preferred_element_type=jnp.float32)
        hs1 = h1 * dinv
        row = lax.broadcasted_iota(jnp.int32, (_BR, 1), 0) + i * _BR
        o_ref[...] = jnp.where(row < _N, hs1, 0.0)

    g = _NP // _BR
    return pl.pallas_call(
        body,
        grid=(g,),
        in_specs=[
            pl.BlockSpec((_BR, 128), lambda i: (i, 0)),
            pl.BlockSpec((_BR, 128), lambda i: (i, 0)),
            pl.BlockSpec((_BR, 128), lambda i: (i, 0)),
            pl.BlockSpec((_BR, 1), lambda i: (i, 0)),
            pl.BlockSpec((1, 128), lambda i: (0, 0)),
            pl.BlockSpec((128, 256), lambda i: (0, 0)),
        ],
        out_specs=pl.BlockSpec((_BR, 256), lambda i: (i, 0)),
        out_shape=jax.ShapeDtypeStruct((_NP, 256), jnp.float32),
    )(acc_a, acc_b, hs0, dinv, b0, w1)


def _tc_c(acc_a, acc_b, hs1, dinv, b1, wl1, bl1, wl2, bl2, wl3, bl3):
    def body(aa_ref, ab_ref, h_ref, d_ref, b_ref, w1_ref, c1_ref, w2_ref,
             c2_ref, w3_ref, c3_ref, o_ref):
        dinv = d_ref[...]
        agg = jnp.concatenate([aa_ref[...], ab_ref[...]], axis=1) + h_ref[...]
        h = jnp.maximum(agg * dinv + b_ref[...], 0.0)
        h = jnp.maximum(
            jnp.dot(h, w1_ref[...], preferred_element_type=jnp.float32)
            + c1_ref[...], 0.0)
        h = jnp.maximum(
            jnp.dot(h, w2_ref[...], preferred_element_type=jnp.float32)
            + c2_ref[...], 0.0)
        h = jnp.maximum(
            jnp.dot(h, w3_ref[...], preferred_element_type=jnp.float32)
            + c3_ref[...], 0.0)
        m = jnp.max(h, axis=1, keepdims=True)
        e = jnp.exp(h - m)
        o_ref[...] = e / jnp.sum(e, axis=1, keepdims=True)

    g = _NP // _BR
    return pl.pallas_call(
        body,
        grid=(g,),
        in_specs=[
            pl.BlockSpec((_BR, 128), lambda i: (i, 0)),
            pl.BlockSpec((_BR, 128), lambda i: (i, 0)),
            pl.BlockSpec((_BR, 256), lambda i: (i, 0)),
            pl.BlockSpec((_BR, 1), lambda i: (i, 0)),
            pl.BlockSpec((1, 256), lambda i: (0, 0)),
            pl.BlockSpec((256, 512), lambda i: (0, 0)),
            pl.BlockSpec((1, 512), lambda i: (0, 0)),
            pl.BlockSpec((512, 1024), lambda i: (0, 0)),
            pl.BlockSpec((1, 1024), lambda i: (0, 0)),
            pl.BlockSpec((1024, 64), lambda i: (0, 0)),
            pl.BlockSpec((1, 64), lambda i: (0, 0)),
        ],
        out_specs=pl.BlockSpec((_BR, 64), lambda i: (i, 0)),
        out_shape=jax.ShapeDtypeStruct((_NP, 64), jnp.float32),
    )(acc_a, acc_b, hs1, dinv, b1, wl1, bl1, wl2, bl2, wl3, bl3)


@jax.jit
def kernel(x, edge_index, W0, b0, W1, b1, Wl1, bl1, Wl2, bl2, Wl3, bl3):
    src = edge_index[0]
    dst = edge_index[1]
    npad = _EP - _E
    pad_idx = _N + (jnp.arange(npad, dtype=jnp.int32) % (_NP - _N))
    tail = jnp.full((_TAIL,), _N, jnp.int32)
    src_p = jnp.concatenate([src, pad_idx])
    dst_p = jnp.concatenate([dst, pad_idx, tail]).reshape(-1, _CHUNK)
    src2 = jnp.concatenate(
        [src_p * 2, src_p * 2 + 1, tail]).reshape(-1, _CHUNK)
    src_p = jnp.concatenate([src_p, tail]).reshape(-1, _CHUNK)

    x_pad = jnp.zeros((_NP, 128), jnp.float32).at[:_N].set(x)

    deg2 = _sc_deg(dst_p)
    deg_a = deg2[0]
    deg_b = deg2[1]

    h0 = _tc_a1(x_pad, W0)
    hs0, dinv = _tc_a2(h0, deg_a, deg_b)
    acc0 = _sc_agg(hs0, src_p, dst_p, edge_split=True)
    hs1 = _tc_b(acc0[0], acc0[1], hs0, dinv, b0.reshape(1, 128), W1)
    acc1 = _sc_agg(hs1.reshape(2 * _NP, 128), src2, dst_p, edge_split=False)
    out = _tc_c(acc1[0], acc1[1], hs1, dinv, b1.reshape(1, 256),
                Wl1, bl1.reshape(1, 512), Wl2, bl2.reshape(1, 1024),
                Wl3, bl3.reshape(1, 64))
    return out[:_N]
